# baseline (speedup 1.0000x reference)
import functools
import jax
import jax.numpy as jnp
from jax import lax
from jax.experimental import pallas as pl
from jax.experimental.pallas import tpu as pltpu
from jax.experimental.pallas import tpu_sc as plsc

VOCAB = 1000000
D = 64
SCALE = 8.0

_info = plsc.get_sparse_core_info()
NC = _info.num_cores
NS = _info.num_subcores
L = _info.num_lanes
NW = NC * NS

B = 4096 * 200
CH = 128
B_PER_W = B // NW
NCHUNK = B_PER_W // CH

_mesh = plsc.VectorSubcoreMesh(core_axis_name="c", subcore_axis_name="s")


@functools.partial(
    pl.kernel,
    mesh=_mesh,
    compiler_params=pltpu.CompilerParams(use_tc_tiling_on_sc=False),
    out_type=jax.ShapeDtypeStruct((NW, NCHUNK, CH, D), jnp.float32),
    scratch_types=[
        pltpu.VMEM((NCHUNK, CH), jnp.int32),
        pltpu.VMEM((CH, D), jnp.float32),
        pltpu.SemaphoreType.DMA,
    ],
)
def _embed_kernel(x_hbm, lut_hbm, out_hbm, idx_v, rows_v, gsem):
    wid = lax.axis_index("s") * NC + lax.axis_index("c")
    pltpu.sync_copy(x_hbm.at[wid], idx_v)

    def do_chunk(j, carry):
        pltpu.async_copy(lut_hbm.at[idx_v.at[j]], rows_v, gsem).wait()

        def mul_row(i, c2):
            for q in range(D // L):
                s = rows_v[i, pl.ds(q * L, L)]
                rows_v[i, pl.ds(q * L, L)] = s * SCALE
            return c2

        lax.fori_loop(0, CH, mul_row, 0)

        pltpu.sync_copy(rows_v, out_hbm.at[wid, j])
        return carry

    lax.fori_loop(0, NCHUNK, do_chunk, 0)


def kernel(x, lut):
    xr = x.astype(jnp.int32).reshape(NW, NCHUNK, CH)
    out = _embed_kernel(xr, lut)
    return out.reshape(x.shape[0], x.shape[1], D)


# speedup vs baseline: 1.2122x; 1.2122x over previous
import functools
import jax
import jax.numpy as jnp
from jax import lax
from jax.experimental import pallas as pl
from jax.experimental.pallas import tpu as pltpu
from jax.experimental.pallas import tpu_sc as plsc

VOCAB = 1000000
D = 64
SCALE = 8.0

_info = plsc.get_sparse_core_info()
NC = _info.num_cores
NS = _info.num_subcores
L = _info.num_lanes
NW = NC * NS

B = 4096 * 200
CH = 128
B_PER_W = B // NW
NCHUNK = B_PER_W // CH
NBUF = 4
NOUTER = NCHUNK // NBUF
RU = 4

_mesh = plsc.VectorSubcoreMesh(core_axis_name="c", subcore_axis_name="s")


@functools.partial(
    pl.kernel,
    mesh=_mesh,
    compiler_params=pltpu.CompilerParams(use_tc_tiling_on_sc=False),
    out_type=jax.ShapeDtypeStruct((NW, NCHUNK, CH, D), jnp.float32),
    scratch_types=[
        pltpu.VMEM((NCHUNK, CH), jnp.int32),
        pltpu.VMEM((NBUF, CH, D), jnp.float32),
        pltpu.VMEM((NBUF, CH, D), jnp.float32),
        pltpu.SemaphoreType.DMA((NBUF,)),
        pltpu.SemaphoreType.DMA((NBUF,)),
    ],
)
def _embed_kernel(x_hbm, lut_hbm, out_hbm, idx_v, gbuf, obuf, gsem, psem):
    wid = lax.axis_index("s") * NC + lax.axis_index("c")
    pltpu.sync_copy(x_hbm.at[wid], idx_v)

    def start_gather(j, b):
        pltpu.make_async_copy(
            lut_hbm.at[idx_v.at[j]], gbuf.at[b], gsem.at[b]).start()

    def wait_gather(j, b):
        pltpu.make_async_copy(
            lut_hbm.at[idx_v.at[j]], gbuf.at[b], gsem.at[b]).wait()

    def start_put(j, b):
        pltpu.make_async_copy(
            obuf.at[b], out_hbm.at[wid, j], psem.at[b]).start()

    def wait_put(j, b):
        pltpu.make_async_copy(
            obuf.at[b], out_hbm.at[wid, j], psem.at[b]).wait()

    def scale_chunk(b):
        def mrow(i, c):
            for r in range(RU):
                ii = i * RU + r
                for q in range(D // L):
                    sl = pl.ds(q * L, L)
                    obuf[b, ii, sl] = gbuf[b, ii, sl] * SCALE
            return c
        lax.fori_loop(0, CH // RU, mrow, 0)

    for b in range(NBUF):
        start_gather(b, b)

    def step(j, b, first, last):
        wait_gather(j, b)
        if not first:
            wait_put(j, b)
        scale_chunk(b)
        if not last:
            start_gather(j + NBUF, b)
        start_put(j, b)

    for b in range(NBUF):
        step(b, b, True, False)

    def outer(g, c):
        j0 = g * NBUF
        for b in range(NBUF):
            step(j0 + b, b, False, False)
        return c

    lax.fori_loop(1, NOUTER - 1, outer, 0)

    for b in range(NBUF):
        step((NOUTER - 1) * NBUF + b, b, False, True)

    for b in range(NBUF):
        wait_put((NOUTER - 1) * NBUF + b, b)


def kernel(x, lut):
    xr = x.astype(jnp.int32).reshape(NW, NCHUNK, CH)
    out = _embed_kernel(xr, lut)
    return out.reshape(x.shape[0], x.shape[1], D)
